# baseline (speedup 1.0000x reference)
import functools

import jax
import jax.numpy as jnp
from jax import lax
from jax.experimental import pallas as pl
from jax.experimental.pallas import tpu as pltpu
from jax.experimental.pallas import tpu_sc as plsc

_EMB_DIM = 64
_EMB_SIZE = 1024
_N = 9216

_R = 1024
_NB = _N // _R

_NC = 2
_NS = 16
_NW = _NC * _NS
_BPW = _N // _NW
_CH = 96
_NCH = _BPW // _CH


def _argmin_body(flat_ref, wt_ref, idx_ref):
    flat = flat_ref[...]
    wt = wt_ref[...]
    scores = lax.dot_general(flat, wt, (((1,), (0,)), ((), ())),
                             preferred_element_type=jnp.float32)
    wsq = jnp.sum(wt * wt, axis=0, keepdims=True)
    xsq = jnp.sum(flat * flat, axis=1, keepdims=True)
    dist = wsq - 2.0 * scores + xsq
    m = jnp.min(dist, axis=1, keepdims=True)
    cols = lax.broadcasted_iota(jnp.int32, dist.shape, 1)
    idx_ref[...] = jnp.min(jnp.where(dist == m, cols, _EMB_SIZE), axis=1,
                           keepdims=True)


def _argmin_indices(flat, Wt):
    return pl.pallas_call(
        _argmin_body,
        grid=(_NB,),
        in_specs=[
            pl.BlockSpec((_R, _EMB_DIM), lambda i: (i, 0)),
            pl.BlockSpec((_EMB_DIM, _EMB_SIZE), lambda i: (0, 0)),
        ],
        out_specs=pl.BlockSpec((_R, 1), lambda i: (i, 0)),
        out_shape=jax.ShapeDtypeStruct((_N, 1), jnp.int32),
    )(flat, Wt)


def _gather_body(table_hbm, idx_hbm, out_hbm, idx_v, rows_v, sem):
    wid = lax.axis_index("s") * _NC + lax.axis_index("c")
    pltpu.sync_copy(idx_hbm.at[wid], idx_v)
    copies = [
        pltpu.async_copy(table_hbm.at[idx_v.at[c]],
                         rows_v.at[pl.ds(c * _CH, _CH)], sem)
        for c in range(_NCH)
    ]
    for cp in copies:
        cp.wait()
    pltpu.sync_copy(rows_v, out_hbm.at[pl.ds(wid * _BPW, _BPW)])


def _gather_rows(Wpad, idx3):
    f = functools.partial(
        pl.kernel,
        out_type=jax.ShapeDtypeStruct((_N, 128), jnp.float32),
        mesh=plsc.VectorSubcoreMesh(core_axis_name="c", subcore_axis_name="s",
                                    num_cores=_NC, num_subcores=_NS),
        scratch_types=[
            pltpu.VMEM((_NCH, _CH), jnp.int32),
            pltpu.VMEM((_BPW, 128), jnp.float32),
            pltpu.SemaphoreType.DMA,
        ],
    )(_gather_body)
    return f(Wpad, idx3)


def kernel(x, W):
    B, T, D = x.shape
    flat = x.reshape(_N, D)
    idx = _argmin_indices(flat, W.T)
    Wpad = jnp.pad(W, ((0, 0), (0, 128 - _EMB_DIM)))
    embed = _gather_rows(Wpad, idx.reshape(_NW, _NCH, _CH))
    embed = embed[:, :_EMB_DIM].reshape(B, T, D)
    return (embed, embed, idx.reshape(B, T))
